# baseline (speedup 1.0000x reference)
import functools
import jax
import jax.numpy as jnp
from jax import lax
from jax.experimental import pallas as pl
from jax.experimental.pallas import tpu as pltpu
from jax.experimental.pallas import tpu_sc as plsc

NBUF = 8


def _make_kernel(B, S, D):
    info = plsc.get_sparse_core_info()
    NC, NS = info.num_cores, info.num_subcores
    NW = NC * NS
    assert B % NW == 0
    rows_per_w = B // NW
    assert rows_per_w % NBUF == 0
    n_outer = rows_per_w // NBUF

    mesh = plsc.VectorSubcoreMesh(core_axis_name="c", subcore_axis_name="s")

    @functools.partial(
        pl.kernel,
        mesh=mesh,
        out_type=jax.ShapeDtypeStruct((B, S, D), jnp.float32),
        scratch_types=(
            [pltpu.VMEM((rows_per_w, S), jnp.int32)]
            + [pltpu.VMEM((S, D), jnp.float32) for _ in range(NBUF)]
            + [pltpu.SemaphoreType.DMA, pltpu.SemaphoreType.DMA]
        ),
    )
    def k(table_hbm, x_hbm, out_hbm, idx_v, *rest):
        bufs = rest[:NBUF]
        sem_g, sem_w = rest[NBUF], rest[NBUF + 1]
        wid = lax.axis_index("s") * NC + lax.axis_index("c")
        base = wid * rows_per_w
        pltpu.sync_copy(x_hbm.at[pl.ds(base, rows_per_w)], idx_v)

        def gather(r, buf):
            return pltpu.async_copy(table_hbm.at[idx_v.at[r]], buf, sem_g)

        def wait_gather(r, buf):
            pltpu.make_async_copy(table_hbm.at[idx_v.at[r]], buf, sem_g).wait()

        def write(r, buf):
            return pltpu.async_copy(buf, out_hbm.at[base + r], sem_w)

        def wait_write(r, buf):
            pltpu.make_async_copy(buf, out_hbm.at[base + r], sem_w).wait()

        def outer(o, carry):
            r0 = o * NBUF
            for b in range(NBUF):
                @pl.when(o > 0)
                def _():
                    wait_write(r0 - NBUF + b, bufs[b])

                gather(r0 + b, bufs[b])
            for b in range(NBUF):
                wait_gather(r0 + b, bufs[b])
                write(r0 + b, bufs[b])
            return carry

        lax.fori_loop(0, n_outer, outer, 0)
        for b in range(NBUF):
            wait_write((n_outer - 1) * NBUF + b, bufs[b])

    return k


NSPLIT = 4


def kernel(x, weight):
    B, S = x.shape
    V, D = weight.shape
    bs = B // NSPLIT
    k = _make_kernel(bs, S, D)
    xi = x.astype(jnp.int32)
    outs = [k(weight, xi[i * bs:(i + 1) * bs]) for i in range(NSPLIT)]
    return jnp.concatenate(outs, axis=0)


# speedup vs baseline: 3.2128x; 3.2128x over previous
import functools
import jax
import jax.numpy as jnp
from jax import lax
from jax.experimental import pallas as pl
from jax.experimental.pallas import tpu as pltpu
from jax.experimental.pallas import tpu_sc as plsc

NBUF = 5


def _make_kernel(B, S, D):
    info = plsc.get_sparse_core_info()
    NC, NS = info.num_cores, info.num_subcores
    NW = NC * NS
    assert B % NW == 0
    cols = B // NW
    assert S % NBUF == 0
    n_outer = S // NBUF

    mesh = plsc.VectorSubcoreMesh(core_axis_name="c", subcore_axis_name="s")

    @functools.partial(
        pl.kernel,
        mesh=mesh,
        out_type=jax.ShapeDtypeStruct((S, B, D), jnp.float32),
        scratch_types=(
            [pltpu.VMEM((S, cols), jnp.int32)]
            + [pltpu.VMEM((cols, D), jnp.float32) for _ in range(NBUF)]
            + [pltpu.SemaphoreType.DMA, pltpu.SemaphoreType.DMA]
        ),
    )
    def k(table_hbm, xt_hbm, out_hbm, idx_v, *rest):
        bufs = rest[:NBUF]
        sem_g, sem_w = rest[NBUF], rest[NBUF + 1]
        wid = lax.axis_index("s") * NC + lax.axis_index("c")
        b0 = wid * cols
        pltpu.sync_copy(xt_hbm.at[:, pl.ds(b0, cols)], idx_v)

        def gather(s, buf):
            return pltpu.async_copy(table_hbm.at[idx_v.at[s]], buf, sem_g)

        def wait_gather(s, buf):
            pltpu.make_async_copy(table_hbm.at[idx_v.at[s]], buf, sem_g).wait()

        def write(s, buf):
            return pltpu.async_copy(
                buf, out_hbm.at[s, pl.ds(b0, cols)], sem_w
            )

        def wait_write(s, buf):
            pltpu.make_async_copy(
                buf, out_hbm.at[s, pl.ds(b0, cols)], sem_w
            ).wait()

        def outer(o, carry):
            s0 = o * NBUF
            for b in range(NBUF):
                @pl.when(o > 0)
                def _():
                    wait_write(s0 - NBUF + b, bufs[b])

                gather(s0 + b, bufs[b])
            for b in range(NBUF):
                wait_gather(s0 + b, bufs[b])
                write(s0 + b, bufs[b])
            return carry

        lax.fori_loop(0, n_outer, outer, 0)
        for b in range(NBUF):
            wait_write((n_outer - 1) * NBUF + b, bufs[b])

    return k


def kernel(x, weight):
    B, S = x.shape
    V, D = weight.shape
    k = _make_kernel(B, S, D)
    xt = jnp.transpose(x.astype(jnp.int32), (1, 0))
    out3 = k(weight, xt)
    return jnp.transpose(out3, (1, 0, 2))
